# baseline (speedup 1.0000x reference)
import functools

import jax
import jax.numpy as jnp
from jax import lax
from jax.experimental import pallas as pl
from jax.experimental.pallas import tpu as pltpu, tpu_sc as plsc

VOCAB = 49408
MAXPOS = 8192
DIM = 1024
NSPECIAL = 16
LANES = 16
NW = 32
ROWS_PER_W = MAXPOS // NW
CHUNK = 16
NBUF = 2
NCHUNK = ROWS_PER_W // CHUNK
VECS_PER_ROW = DIM // LANES
CVE = CHUNK * DIM

_mesh = plsc.VectorSubcoreMesh(core_axis_name="c", subcore_axis_name="s")


@functools.partial(
    pl.kernel,
    out_type=jax.ShapeDtypeStruct(((MAXPOS + NSPECIAL) * DIM,), jnp.float32),
    mesh=_mesh,
    scratch_types=[
        pltpu.VMEM((ROWS_PER_W,), jnp.int32),
        pltpu.VMEM((NBUF, CHUNK, DIM), jnp.float32),
        pltpu.VMEM((NBUF * CVE,), jnp.float32),
        pltpu.VMEM((NSPECIAL * DIM,), jnp.float32),
        [pltpu.SemaphoreType.DMA] * NBUF,
        [pltpu.SemaphoreType.DMA] * NBUF,
        [pltpu.SemaphoreType.DMA] * NBUF,
        pltpu.SemaphoreType.DMA,
    ],
)
def _embed_kernel(ids_hbm, tok_hbm, pos_hbm, spec_hbm, out_hbm,
                  idx_v, tok_v, pos_v, spec_v, gsem, psem, osem, ssem):
    wid = lax.axis_index("s") * 2 + lax.axis_index("c")
    base = wid * ROWS_PER_W
    pltpu.sync_copy(ids_hbm.at[pl.ds(NSPECIAL + base, ROWS_PER_W)], idx_v)

    @pl.when(wid == 0)
    def _():
        pltpu.sync_copy(spec_hbm, spec_v)
        pltpu.async_copy(spec_v, out_hbm.at[pl.ds(DIM, NSPECIAL * DIM)], ssem)

    def issue_in(g, b):
        cbase = g * CHUNK
        pltpu.async_copy(
            tok_hbm.at[idx_v.at[pl.ds(cbase, CHUNK)]], tok_v.at[b], gsem[b])
        pltpu.async_copy(
            pos_hbm.at[pl.ds((base + cbase) * DIM, CVE)],
            pos_v.at[pl.ds(b * CVE, CVE)], psem[b])

    def wait_in(b):
        pltpu.make_async_copy(
            tok_hbm.at[pl.ds(0, CHUNK)], tok_v.at[b], gsem[b]).wait()
        pltpu.make_async_copy(
            pos_hbm.at[pl.ds(0, CVE)],
            pos_v.at[pl.ds(b * CVE, CVE)], psem[b]).wait()

    def issue_out(g, b):
        first = jnp.logical_and(wid == 0, g == 0)

        @pl.when(first)
        def _():
            pltpu.async_copy(
                pos_v.at[pl.ds(b * CVE, DIM)],
                out_hbm.at[pl.ds(0, DIM)], osem[b])
            pltpu.async_copy(
                pos_v.at[pl.ds(b * CVE + DIM, CVE - DIM)],
                out_hbm.at[pl.ds((NSPECIAL + 1) * DIM, CVE - DIM)], osem[b])

        @pl.when(jnp.logical_not(first))
        def _():
            pltpu.async_copy(
                pos_v.at[pl.ds(b * CVE, CVE)],
                out_hbm.at[pl.ds((NSPECIAL + base + g * CHUNK) * DIM, CVE)],
                osem[b])

    def wait_out(g, b):
        first = jnp.logical_and(wid == 0, g == 0)

        @pl.when(first)
        def _():
            pltpu.make_async_copy(
                pos_v.at[pl.ds(b * CVE, DIM)],
                out_hbm.at[pl.ds(0, DIM)], osem[b]).wait()
            pltpu.make_async_copy(
                pos_v.at[pl.ds(0, CVE - DIM)],
                out_hbm.at[pl.ds(0, CVE - DIM)], osem[b]).wait()

        @pl.when(jnp.logical_not(first))
        def _():
            pltpu.make_async_copy(
                pos_v.at[pl.ds(0, CVE)],
                out_hbm.at[pl.ds(0, CVE)], osem[b]).wait()

    issue_in(0, 0)

    def outer(i, carry):
        for b in range(NBUF):
            g = i * NBUF + b
            nb = 1 - b

            @pl.when(g >= 1)
            def _():
                wait_out(g - 1, nb)

            @pl.when(g + 1 < NCHUNK)
            def _():
                issue_in(g + 1, nb)

            wait_in(b)

            def row_body(r, carry2):
                for c in range(VECS_PER_ROW):
                    plsc.addupdate(
                        pos_v.at[pl.ds(b * CVE + r * DIM + c * LANES, LANES)],
                        tok_v[b, r, pl.ds(c * LANES, LANES)])
                return carry2
            lax.fori_loop(0, CHUNK, row_body, 0)

            issue_out(g, b)
        return carry

    lax.fori_loop(0, NCHUNK // NBUF, outer, 0)
    wait_out(NCHUNK - 1, (NCHUNK - 1) % NBUF)

    @pl.when(wid == 0)
    def _():
        pltpu.make_async_copy(
            spec_v, out_hbm.at[pl.ds(DIM, NSPECIAL * DIM)], ssem).wait()


def kernel(input_ids, token_table, pos_table, special_token_embedding):
    ids_flat = input_ids.reshape(MAXPOS + NSPECIAL)
    pos_flat = pos_table.reshape(MAXPOS * DIM)
    spec = special_token_embedding.reshape(NSPECIAL * DIM)
    out = _embed_kernel(ids_flat, token_table, pos_flat, spec)
    return out.reshape(1, MAXPOS + NSPECIAL, DIM)


# speedup vs baseline: 2.1331x; 2.1331x over previous
import functools

import jax
import jax.numpy as jnp
from jax import lax
from jax.experimental import pallas as pl
from jax.experimental.pallas import tpu as pltpu, tpu_sc as plsc

VOCAB = 49408
MAXPOS = 8192
DIM = 1024
NSPECIAL = 16
LANES = 16
NW = 32
ROWS_PER_W = MAXPOS // NW
CHUNK = 16
NBUF = 2
NCHUNK = ROWS_PER_W // CHUNK
VECS_PER_ROW = DIM // LANES

_mesh = plsc.VectorSubcoreMesh(core_axis_name="c", subcore_axis_name="s")


@functools.partial(
    pl.kernel,
    out_type=jax.ShapeDtypeStruct((MAXPOS + NSPECIAL, DIM), jnp.float32),
    mesh=_mesh,
    scratch_types=[
        pltpu.VMEM((ROWS_PER_W,), jnp.int32),
        pltpu.VMEM((NBUF, CHUNK, DIM), jnp.float32),
        pltpu.VMEM((NBUF, CHUNK, DIM), jnp.float32),
        pltpu.VMEM((8 + NSPECIAL, DIM), jnp.float32),
        pltpu.VMEM((NSPECIAL, DIM), jnp.float32),
        pltpu.VMEM((8, DIM), jnp.float32),
        [pltpu.SemaphoreType.DMA] * NBUF,
        [pltpu.SemaphoreType.DMA] * NBUF,
        [pltpu.SemaphoreType.DMA] * NBUF,
    ],
)
def _embed_kernel(ids_hbm, tok_hbm, pos_hbm, spec_hbm, out_hbm,
                  idx_v, tok_v, pos_v, head_v, spec_v, tmp8_v,
                  gsem, psem, osem):
    wid = lax.axis_index("s") * 2 + lax.axis_index("c")
    base = wid * ROWS_PER_W
    pltpu.sync_copy(ids_hbm.at[pl.ds(NSPECIAL + base, ROWS_PER_W)], idx_v)

    def issue_in(g, b):
        cbase = g * CHUNK
        pltpu.async_copy(
            tok_hbm.at[idx_v.at[pl.ds(cbase, CHUNK)]], tok_v.at[b], gsem[b])
        pltpu.async_copy(
            pos_hbm.at[pl.ds(base + cbase, CHUNK)], pos_v.at[b], psem[b])

    def wait_in(b):
        pltpu.make_async_copy(
            tok_hbm.at[pl.ds(0, CHUNK)], tok_v.at[b], gsem[b]).wait()
        pltpu.make_async_copy(
            pos_hbm.at[pl.ds(0, CHUNK)], pos_v.at[b], psem[b]).wait()

    def issue_out(g, b):
        pltpu.async_copy(
            pos_v.at[b],
            out_hbm.at[pl.ds(NSPECIAL + base + g * CHUNK, CHUNK)], osem[b])

    def wait_out(b):
        pltpu.make_async_copy(
            pos_v.at[b], out_hbm.at[pl.ds(NSPECIAL, CHUNK)], osem[b]).wait()

    def head_repair():
        pltpu.sync_copy(out_hbm.at[pl.ds(NSPECIAL, 8)], tmp8_v)
        pltpu.sync_copy(spec_hbm, spec_v)

        def copy_spec(k, carry2):
            for c in range(VECS_PER_ROW):
                sl = pl.ds(c * LANES, LANES)
                head_v[1 + k, sl] = spec_v[k, sl]
            return carry2
        lax.fori_loop(0, NSPECIAL, copy_spec, 0)

        def copy_sub(j, carry2):
            for c in range(VECS_PER_ROW):
                sl = pl.ds(c * LANES, LANES)
                head_v[jnp.where(j == 0, 0, NSPECIAL + j), sl] = tmp8_v[j, sl]
            return carry2
        lax.fori_loop(0, 8, copy_sub, 0)

        pltpu.sync_copy(head_v, out_hbm.at[pl.ds(0, 8 + NSPECIAL)])

    issue_in(0, 0)

    def outer(i, carry):
        for b in range(NBUF):
            g = i * NBUF + b
            nb = 1 - b

            @pl.when(g >= 1)
            def _():
                wait_out(nb)

            @pl.when(g + 1 < NCHUNK)
            def _():
                issue_in(g + 1, nb)

            wait_in(b)

            def row_body(r, carry2):
                for c in range(VECS_PER_ROW):
                    sl = pl.ds(c * LANES, LANES)
                    plsc.addupdate(pos_v.at[b, r, sl], tok_v[b, r, sl])
                return carry2
            lax.fori_loop(0, CHUNK, row_body, 0)

            issue_out(g, b)

            @pl.when(jnp.logical_and(wid == 0, g == 1))
            def _():
                head_repair()
        return carry

    lax.fori_loop(0, NCHUNK // NBUF, outer, 0)
    wait_out((NCHUNK - 1) % NBUF)


def kernel(input_ids, token_table, pos_table, special_token_embedding):
    ids_flat = input_ids.reshape(MAXPOS + NSPECIAL)
    spec = special_token_embedding.reshape(NSPECIAL, DIM)
    out = _embed_kernel(ids_flat, token_table, pos_table, spec)
    return out.reshape(1, MAXPOS + NSPECIAL, DIM)
